# baseline (speedup 1.0000x reference)
import functools

import jax
import jax.numpy as jnp
import numpy as np
from jax import lax
from jax.experimental import pallas as pl
from jax.experimental.pallas import tpu as pltpu
from jax.experimental.pallas import tpu_sc as plsc

N_NODES = 10000
N_EDGES = 320000
D = 128
DH = D // 2

NC = 2
NS = 16
NW = NC * NS
CHUNK = 40

NSLICE = 10
SLICE_E = N_EDGES // NSLICE
EPW_S = SLICE_E // NW
NCHUNKS_S = EPW_S // CHUNK
EBLK = 1600
CBLKS = SLICE_E // EBLK

_DN = (((1,), (1,)), ((), ()))

_PIDX = np.arange(D).reshape(D // 32, 2, 16)
PERM_LO = _PIDX[:, 0, :].ravel()
PERM_HI = _PIDX[:, 1, :].ravel()


def _node_proj_body(nf_ref, ws_ref, wd_ref, b1_ref, ts_ref, td_ref):
    nf = nf_ref[...]
    ts_ref[...] = lax.dot_general(nf, ws_ref[...], _DN, preferred_element_type=jnp.float32)
    td_ref[...] = (
        lax.dot_general(nf, wd_ref[...], _DN, preferred_element_type=jnp.float32)
        + b1_ref[...]
    )


def _node_proj(node_feats, W_src, W_dst, b1):
    nblk = 2000
    grid = (N_NODES // nblk,)
    whole = lambda i: (0, 0)
    return pl.pallas_call(
        _node_proj_body,
        grid=grid,
        in_specs=[
            pl.BlockSpec((nblk, D), lambda i: (i, 0)),
            pl.BlockSpec((D, D), whole),
            pl.BlockSpec((D, D), whole),
            pl.BlockSpec((1, D), whole),
        ],
        out_specs=[
            pl.BlockSpec((nblk, D), lambda i: (i, 0)),
            pl.BlockSpec((nblk, D), lambda i: (i, 0)),
        ],
        out_shape=[
            jax.ShapeDtypeStruct((N_NODES, D), jnp.float32),
            jax.ShapeDtypeStruct((N_NODES, D), jnp.float32),
        ],
    )(node_feats, W_src, W_dst, b1)


def _rne_bf16_bits(s_bits):
    return lax.shift_right_logical(
        s_bits + jnp.int32(0x7FFF)
        + lax.bitwise_and(lax.shift_right_logical(s_bits, 16), jnp.int32(1)),
        16)


def _gather_sum_body(sbase, ts_hbm, td_hbm, src_hbm, dst_hbm, out_hbm,
                     idx_sa, idx_da, rows_s, rows_d, rows_o, sem_s, sem_d):
    wid = lax.axis_index("s") * NC + lax.axis_index("c")
    wbase = wid * EPW_S

    pltpu.sync_copy(src_hbm.at[pl.ds(sbase + wbase, EPW_S)], idx_sa)
    pltpu.sync_copy(dst_hbm.at[pl.ds(sbase + wbase, EPW_S)], idx_da)

    def issue(c, b):
        off = pl.multiple_of(c * CHUNK, 8)
        pltpu.async_copy(ts_hbm.at[idx_sa.at[pl.ds(off, CHUNK)]], rows_s[b], sem_s[b])
        pltpu.async_copy(td_hbm.at[idx_da.at[pl.ds(off, CHUNK)]], rows_d[b], sem_d[b])

    def process(c, b):
        off = pl.multiple_of(c * CHUNK, 8)
        pltpu.make_async_copy(
            ts_hbm.at[idx_sa.at[pl.ds(off, CHUNK)]], rows_s[b], sem_s[b]).wait()
        pltpu.make_async_copy(
            td_hbm.at[idx_da.at[pl.ds(off, CHUNK)]], rows_d[b], sem_d[b]).wait()

        def add_body(e, acc):
            for j in range(D // 32):
                lo = (e, pl.ds(j * 32, 16))
                hi = (e, pl.ds(j * 32 + 16, 16))
                s_lo = lax.bitcast_convert_type(
                    rows_s[b][lo] + rows_d[b][lo], jnp.int32)
                s_hi = lax.bitcast_convert_type(
                    rows_s[b][hi] + rows_d[b][hi], jnp.int32)
                rows_o[b][e, pl.ds(j * 16, 16)] = lax.bitwise_or(
                    _rne_bf16_bits(s_lo),
                    lax.shift_left(_rne_bf16_bits(s_hi), 16))
            return acc

        lax.fori_loop(0, CHUNK, add_body, 0)
        pltpu.sync_copy(rows_o[b], out_hbm.at[pl.ds(wbase + c * CHUNK, CHUNK)])

    issue(0, 0)

    def pair_body(p, carry):
        for b in range(2):
            c = 2 * p + b
            nxt = c + 1

            @pl.when(nxt < NCHUNKS_S)
            def _():
                issue(nxt, 1 - b)

            @pl.when(c < NCHUNKS_S)
            def _():
                process(c, b)
        return carry

    lax.fori_loop(0, (NCHUNKS_S + 1) // 2, pair_body, 0)


def _gather_sum(t_src, t_dst, src, dst, sbase):
    mesh = plsc.VectorSubcoreMesh(
        core_axis_name="c", subcore_axis_name="s", num_cores=NC, num_subcores=NS
    )
    fn = pl.kernel(
        functools.partial(_gather_sum_body, sbase),
        out_type=jax.ShapeDtypeStruct((SLICE_E, DH), jnp.int32),
        mesh=mesh,
        scratch_types=[
            pltpu.VMEM((EPW_S,), jnp.int32),
            pltpu.VMEM((EPW_S,), jnp.int32),
            [pltpu.VMEM((CHUNK, D), jnp.float32) for _ in range(2)],
            [pltpu.VMEM((CHUNK, D), jnp.float32) for _ in range(2)],
            [pltpu.VMEM((CHUNK, DH), jnp.int32) for _ in range(2)],
            [pltpu.SemaphoreType.DMA for _ in range(2)],
            [pltpu.SemaphoreType.DMA for _ in range(2)],
        ],
    )
    return fn(t_src, t_dst, src, dst)


def _edge_mlp_compute(ef_ref, g_ref, we_ref, wo_ref, bo_ref, out_ref):
    ef = ef_ref[...]
    me = lax.dot_general(ef, we_ref[...], _DN, preferred_element_type=jnp.float32)
    g_bits = g_ref[...]
    g_l = lax.bitcast_convert_type(lax.shift_left(g_bits, 16), jnp.float32)
    g_h = lax.bitcast_convert_type(
        lax.bitwise_and(g_bits, jnp.int32(-65536)), jnp.float32)
    g_cat = jnp.concatenate([g_l, g_h], axis=1)
    h = jnp.maximum(me + g_cat, 0.0)
    out = lax.dot_general(h, wo_ref[...], _DN, preferred_element_type=jnp.float32)
    out_ref[...] = out + bo_ref[...]


def _edge_mlp_body(ef_ref, g_ref, we_ref, wo_ref, bo_ref, out_ref):
    _edge_mlp_compute(ef_ref, g_ref, we_ref, wo_ref, bo_ref, out_ref)


def _edge_mlp_body_acc(ef_ref, g_ref, we_ref, wo_ref, bo_ref, acc_ref, out_ref):
    del acc_ref
    _edge_mlp_compute(ef_ref, g_ref, we_ref, wo_ref, bo_ref, out_ref)


def _edge_mlp_slice(edge_feats, g, W_ecat, Wo_cat, b_out, s, acc):
    whole = lambda i: (0, 0)
    in_specs = [
        pl.BlockSpec((EBLK, D), lambda i, s=s: (s * CBLKS + i, 0)),
        pl.BlockSpec((EBLK, DH), lambda i: (i, 0)),
        pl.BlockSpec((D, D), whole),
        pl.BlockSpec((D, D), whole),
        pl.BlockSpec((1, D), whole),
    ]
    args = [edge_feats, g, W_ecat, Wo_cat, b_out]
    kwargs = {}
    if acc is None:
        body = _edge_mlp_body
    else:
        body = _edge_mlp_body_acc
        in_specs.append(pl.BlockSpec((8, D), whole))
        args.append(acc)
        kwargs["input_output_aliases"] = {5: 0}
    return pl.pallas_call(
        body,
        grid=(CBLKS,),
        in_specs=in_specs,
        out_specs=pl.BlockSpec((EBLK, D), lambda i, s=s: (s * CBLKS + i, 0)),
        out_shape=jax.ShapeDtypeStruct((N_EDGES, D), jnp.float32),
        compiler_params=pltpu.CompilerParams(
            dimension_semantics=("arbitrary",),
        ),
        **kwargs,
    )(*args)


def kernel(edge_feats, node_feats, edge_index, W_edge, W_src, W_dst, b1, W_out, b_out):
    src = edge_index[0].astype(jnp.int32)
    dst = edge_index[1].astype(jnp.int32)
    t_src, t_dst = _node_proj(node_feats, W_src, W_dst, b1.reshape(1, D))
    W_ecat = jnp.concatenate([W_edge[PERM_LO], W_edge[PERM_HI]], axis=0)
    Wo_cat = jnp.concatenate([W_out[:, PERM_LO], W_out[:, PERM_HI]], axis=1)
    bo = b_out.reshape(1, D)
    gs = [_gather_sum(t_src, t_dst, src, dst, s * SLICE_E) for s in range(NSLICE)]
    out = _edge_mlp_slice(edge_feats, gs[0], W_ecat, Wo_cat, bo, 0, None)
    for s in range(1, NSLICE):
        out = _edge_mlp_slice(edge_feats, gs[s], W_ecat, Wo_cat, bo, s, out)
    return out


# speedup vs baseline: 1.1057x; 1.1057x over previous
import functools

import jax
import jax.numpy as jnp
import numpy as np
from jax import lax
from jax.experimental import pallas as pl
from jax.experimental.pallas import tpu as pltpu
from jax.experimental.pallas import tpu_sc as plsc

N_NODES = 10000
N_EDGES = 320000
D = 128
DH = D // 2

NC = 2
NS = 16
NW = NC * NS
CHUNK = 80

NSLICE = 5
SLICE_E = N_EDGES // NSLICE
EPW_S = SLICE_E // NW
NCHUNKS_S = EPW_S // CHUNK
EBLK = 2560
CBLKS = SLICE_E // EBLK

_DN = (((1,), (1,)), ((), ()))

_PIDX = np.arange(D).reshape(D // 32, 2, 16)
PERM_LO = _PIDX[:, 0, :].ravel()
PERM_HI = _PIDX[:, 1, :].ravel()


def _node_proj_body(nf_ref, ws_ref, wd_ref, b1_ref, ts_ref, td_ref):
    nf = nf_ref[...]
    ts_ref[...] = lax.dot_general(nf, ws_ref[...], _DN, preferred_element_type=jnp.float32)
    td_ref[...] = (
        lax.dot_general(nf, wd_ref[...], _DN, preferred_element_type=jnp.float32)
        + b1_ref[...]
    )


def _node_proj(node_feats, W_src, W_dst, b1):
    nblk = 2000
    grid = (N_NODES // nblk,)
    whole = lambda i: (0, 0)
    return pl.pallas_call(
        _node_proj_body,
        grid=grid,
        in_specs=[
            pl.BlockSpec((nblk, D), lambda i: (i, 0)),
            pl.BlockSpec((D, D), whole),
            pl.BlockSpec((D, D), whole),
            pl.BlockSpec((1, D), whole),
        ],
        out_specs=[
            pl.BlockSpec((nblk, D), lambda i: (i, 0)),
            pl.BlockSpec((nblk, D), lambda i: (i, 0)),
        ],
        out_shape=[
            jax.ShapeDtypeStruct((N_NODES, D), jnp.float32),
            jax.ShapeDtypeStruct((N_NODES, D), jnp.float32),
        ],
    )(node_feats, W_src, W_dst, b1)


def _rne_bf16_bits(s_bits):
    return lax.shift_right_logical(
        s_bits + jnp.int32(0x7FFF)
        + lax.bitwise_and(lax.shift_right_logical(s_bits, 16), jnp.int32(1)),
        16)


def _gather_sum_body(sbase, ts_hbm, td_hbm, src_hbm, dst_hbm, out_hbm,
                     idx_sa, idx_da, rows_s, rows_d, rows_o, sem_s, sem_d):
    wid = lax.axis_index("s") * NC + lax.axis_index("c")
    wbase = wid * EPW_S

    pltpu.sync_copy(src_hbm.at[pl.ds(sbase + wbase, EPW_S)], idx_sa)
    pltpu.sync_copy(dst_hbm.at[pl.ds(sbase + wbase, EPW_S)], idx_da)

    def issue(c, b):
        off = pl.multiple_of(c * CHUNK, 8)
        pltpu.async_copy(ts_hbm.at[idx_sa.at[pl.ds(off, CHUNK)]], rows_s[b], sem_s[b])
        pltpu.async_copy(td_hbm.at[idx_da.at[pl.ds(off, CHUNK)]], rows_d[b], sem_d[b])

    def process(c, b):
        off = pl.multiple_of(c * CHUNK, 8)
        pltpu.make_async_copy(
            ts_hbm.at[idx_sa.at[pl.ds(off, CHUNK)]], rows_s[b], sem_s[b]).wait()
        pltpu.make_async_copy(
            td_hbm.at[idx_da.at[pl.ds(off, CHUNK)]], rows_d[b], sem_d[b]).wait()

        def add_body(e, acc):
            for j in range(D // 32):
                lo = (e, pl.ds(j * 32, 16))
                hi = (e, pl.ds(j * 32 + 16, 16))
                s_lo = lax.bitcast_convert_type(
                    rows_s[b][lo] + rows_d[b][lo], jnp.int32)
                s_hi = lax.bitcast_convert_type(
                    rows_s[b][hi] + rows_d[b][hi], jnp.int32)
                rows_o[b][e, pl.ds(j * 16, 16)] = lax.bitwise_or(
                    _rne_bf16_bits(s_lo),
                    lax.shift_left(_rne_bf16_bits(s_hi), 16))
            return acc

        lax.fori_loop(0, CHUNK, add_body, 0)
        pltpu.sync_copy(rows_o[b], out_hbm.at[pl.ds(wbase + c * CHUNK, CHUNK)])

    issue(0, 0)

    def pair_body(p, carry):
        for b in range(2):
            c = 2 * p + b
            nxt = c + 1

            @pl.when(nxt < NCHUNKS_S)
            def _():
                issue(nxt, 1 - b)

            @pl.when(c < NCHUNKS_S)
            def _():
                process(c, b)
        return carry

    lax.fori_loop(0, (NCHUNKS_S + 1) // 2, pair_body, 0)


def _gather_sum(t_src, t_dst, src, dst, sbase):
    mesh = plsc.VectorSubcoreMesh(
        core_axis_name="c", subcore_axis_name="s", num_cores=NC, num_subcores=NS
    )
    fn = pl.kernel(
        functools.partial(_gather_sum_body, sbase),
        out_type=jax.ShapeDtypeStruct((SLICE_E, DH), jnp.int32),
        mesh=mesh,
        scratch_types=[
            pltpu.VMEM((EPW_S,), jnp.int32),
            pltpu.VMEM((EPW_S,), jnp.int32),
            [pltpu.VMEM((CHUNK, D), jnp.float32) for _ in range(2)],
            [pltpu.VMEM((CHUNK, D), jnp.float32) for _ in range(2)],
            [pltpu.VMEM((CHUNK, DH), jnp.int32) for _ in range(2)],
            [pltpu.SemaphoreType.DMA for _ in range(2)],
            [pltpu.SemaphoreType.DMA for _ in range(2)],
        ],
    )
    return fn(t_src, t_dst, src, dst)


def _edge_mlp_compute(ef_ref, g_ref, we_ref, wo_ref, bo_ref, out_ref):
    ef = ef_ref[...]
    me = lax.dot_general(ef, we_ref[...], _DN, preferred_element_type=jnp.float32)
    g_bits = g_ref[...]
    g_l = lax.bitcast_convert_type(lax.shift_left(g_bits, 16), jnp.float32)
    g_h = lax.bitcast_convert_type(
        lax.bitwise_and(g_bits, jnp.int32(-65536)), jnp.float32)
    g_cat = jnp.concatenate([g_l, g_h], axis=1)
    h = jnp.maximum(me + g_cat, 0.0)
    out = lax.dot_general(h, wo_ref[...], _DN, preferred_element_type=jnp.float32)
    out_ref[...] = out + bo_ref[...]


def _edge_mlp_body(ef_ref, g_ref, we_ref, wo_ref, bo_ref, out_ref):
    _edge_mlp_compute(ef_ref, g_ref, we_ref, wo_ref, bo_ref, out_ref)


def _edge_mlp_body_acc(ef_ref, g_ref, we_ref, wo_ref, bo_ref, acc_ref, out_ref):
    del acc_ref
    _edge_mlp_compute(ef_ref, g_ref, we_ref, wo_ref, bo_ref, out_ref)


def _edge_mlp_slice(edge_feats, g, W_ecat, Wo_cat, b_out, s, acc):
    whole = lambda i: (0, 0)
    in_specs = [
        pl.BlockSpec((EBLK, D), lambda i, s=s: (s * CBLKS + i, 0)),
        pl.BlockSpec((EBLK, DH), lambda i: (i, 0)),
        pl.BlockSpec((D, D), whole),
        pl.BlockSpec((D, D), whole),
        pl.BlockSpec((1, D), whole),
    ]
    args = [edge_feats, g, W_ecat, Wo_cat, b_out]
    kwargs = {}
    if acc is None:
        body = _edge_mlp_body
    else:
        body = _edge_mlp_body_acc
        in_specs.append(pl.BlockSpec((8, D), whole))
        args.append(acc)
        kwargs["input_output_aliases"] = {5: 0}
    return pl.pallas_call(
        body,
        grid=(CBLKS,),
        in_specs=in_specs,
        out_specs=pl.BlockSpec((EBLK, D), lambda i, s=s: (s * CBLKS + i, 0)),
        out_shape=jax.ShapeDtypeStruct((N_EDGES, D), jnp.float32),
        compiler_params=pltpu.CompilerParams(
            dimension_semantics=("arbitrary",),
        ),
        **kwargs,
    )(*args)


def kernel(edge_feats, node_feats, edge_index, W_edge, W_src, W_dst, b1, W_out, b_out):
    src = edge_index[0].astype(jnp.int32)
    dst = edge_index[1].astype(jnp.int32)
    t_src, t_dst = _node_proj(node_feats, W_src, W_dst, b1.reshape(1, D))
    W_ecat = jnp.concatenate([W_edge[PERM_LO], W_edge[PERM_HI]], axis=0)
    Wo_cat = jnp.concatenate([W_out[:, PERM_LO], W_out[:, PERM_HI]], axis=1)
    bo = b_out.reshape(1, D)
    gs = [_gather_sum(t_src, t_dst, src, dst, s * SLICE_E) for s in range(NSLICE)]
    out = _edge_mlp_slice(edge_feats, gs[0], W_ecat, Wo_cat, bo, 0, None)
    for s in range(1, NSLICE):
        out = _edge_mlp_slice(edge_feats, gs[s], W_ecat, Wo_cat, bo, s, out)
    return out
